# baseline (speedup 1.0000x reference)
import functools

import jax
import jax.numpy as jnp
from jax import lax
from jax.experimental import pallas as pl
from jax.experimental.pallas import tpu as pltpu
from jax.experimental.pallas import tpu_sc as plsc

N = 10000
E = 320000
IN_CH = 128
HID = 128
OUT = 64

NC = 2
NS = 16
NW = NC * NS
K = 128
NCH = 2 * (-(-E // (2 * NW * K)))
NCHG = NCH + 2
EPAD = NW * K * NCH
RPT = 640
NPAD = NS * RPT


def _make_seg(D, with_cnt):
  mesh = plsc.VectorSubcoreMesh(core_axis_name="c", subcore_axis_name="s")
  out_type = [jax.ShapeDtypeStruct((NC, NPAD, D), jnp.float32)]
  scratch = [
      pltpu.VMEM((K,), jnp.int32),
      pltpu.VMEM((K,), jnp.int32),
      pltpu.VMEM((K,), jnp.int32),
      pltpu.VMEM((K,), jnp.int32),
      pltpu.VMEM((K, D), jnp.float32),
      pltpu.VMEM((K, D), jnp.float32),
      pltpu.VMEM_SHARED((NPAD, D), jnp.float32),
      pltpu.SemaphoreType.DMA,
      pltpu.SemaphoreType.DMA,
  ]
  if with_cnt:
    out_type.append(jax.ShapeDtypeStruct((NC, NS, RPT), jnp.float32))
    scratch += [
        pltpu.VMEM_SHARED((NPAD,), jnp.float32),
        pltpu.VMEM((RPT,), jnp.float32),
        pltpu.VMEM((K,), jnp.float32),
    ]

  def body(p_hbm, src_hbm, dst_hbm, *rest):
    if with_cnt:
      (acc_out, cnt_out, src0_v, src1_v, dst0_v, dst1_v, buf0, buf1,
       acc_sh, sem0, sem1, cnt_sh, zrow_v, ones_v) = rest
    else:
      (acc_out, src0_v, src1_v, dst0_v, dst1_v, buf0, buf1,
       acc_sh, sem0, sem1) = rest
    cid = lax.axis_index("c")
    sid = lax.axis_index("s")
    wid = cid * NS + sid
    r0 = sid * RPT

    zv = jnp.zeros((16,), jnp.float32)

    def zb(i, carry):
      for l in range(D // 16):
        buf0[i, pl.ds(l * 16, 16)] = zv
      return carry
    lax.fori_loop(0, K, zb, 0)
    if with_cnt:
      def cb(i, carry):
        zrow_v[pl.ds(i * 16, 16)] = zv
        return carry
      lax.fori_loop(0, RPT // 16, cb, 0)
      def ob(i, carry):
        ones_v[pl.ds(i * 16, 16)] = zv + 1.0
        return carry
      lax.fori_loop(0, K // 16, ob, 0)
      pltpu.sync_copy(zrow_v, cnt_sh.at[pl.ds(r0, RPT)])

    off = 0
    while off < RPT:
      seg = min(K, RPT - off)
      pltpu.sync_copy(buf0.at[pl.ds(0, seg)], acc_sh.at[pl.ds(r0 + off, seg)])
      off += seg

    pltpu.sync_copy(src_hbm.at[wid, 0], src0_v)
    pltpu.sync_copy(dst_hbm.at[wid, 0], dst0_v)
    pltpu.sync_copy(src_hbm.at[wid, 1], src1_v)
    pltpu.sync_copy(dst_hbm.at[wid, 1], dst1_v)
    pltpu.async_copy(p_hbm.at[src0_v], buf0, sem0)
    pltpu.async_copy(p_hbm.at[src1_v], buf1, sem1)
    plsc.subcore_barrier()

    def eb(g, carry):
      for par, buf, gsem, sv, dv in ((0, buf0, sem0, src0_v, dst0_v),
                                     (1, buf1, sem1, src1_v, dst1_v)):
        j = 2 * g + par
        pltpu.make_async_copy(p_hbm.at[sv], buf, gsem).wait()
        pltpu.sync_copy(buf, acc_sh.at[dv], add=True)
        if with_cnt:
          pltpu.sync_copy(ones_v, cnt_sh.at[dv], add=True)
        pltpu.sync_copy(src_hbm.at[wid, j + 2], sv)
        pltpu.sync_copy(dst_hbm.at[wid, j + 2], dv)
        pltpu.async_copy(p_hbm.at[sv], buf, gsem)
      return carry
    lax.fori_loop(0, NCH // 2, eb, 0)
    pltpu.make_async_copy(p_hbm.at[src0_v], buf0, sem0).wait()
    pltpu.make_async_copy(p_hbm.at[src1_v], buf1, sem1).wait()
    plsc.subcore_barrier()

    pltpu.sync_copy(acc_sh.at[pl.ds(r0, RPT)], acc_out.at[cid, pl.ds(r0, RPT)])
    if with_cnt:
      pltpu.sync_copy(cnt_sh.at[pl.ds(r0, RPT)], cnt_out.at[cid, sid])

  return pl.kernel(body, out_type=tuple(out_type), mesh=mesh,
                   scratch_types=tuple(scratch),
                   compiler_params=pltpu.CompilerParams(
                       use_tc_tiling_on_sc=False))


_seg_cnt = _make_seg(HID, True)
_seg2 = _make_seg(OUT, False)


def _mm1_body(x_ref, wl_ref, wr_ref, b_ref, p_ref, r_ref):
  x = x_ref[...]
  p_ref[...] = jnp.dot(x, wl_ref[...], preferred_element_type=jnp.float32)
  r_ref[...] = jnp.dot(x, wr_ref[...],
                       preferred_element_type=jnp.float32) + b_ref[...]


_mm1 = pl.pallas_call(
    _mm1_body,
    out_shape=(jax.ShapeDtypeStruct((N, HID), jnp.float32),
               jax.ShapeDtypeStruct((N, HID), jnp.float32)),
)


def _mid_body(acc_ref, cnt_ref, r1_ref, wl_ref, wr_ref, b_ref, p2_ref, r2_ref):
  agg = acc_ref[0, :N, :] + acc_ref[1, :N, :]
  cnt = jnp.sum(cnt_ref[:, :N], axis=0)[:, None]
  rc = 1.0 / jnp.maximum(cnt, 1.0)
  h = jnp.maximum(agg * rc + r1_ref[...], 0.0)
  p2_ref[...] = jnp.dot(h, wl_ref[...], preferred_element_type=jnp.float32)
  r2_ref[...] = jnp.dot(h, wr_ref[...],
                        preferred_element_type=jnp.float32) + b_ref[...]


_mid = pl.pallas_call(
    _mid_body,
    out_shape=(jax.ShapeDtypeStruct((N, OUT), jnp.float32),
               jax.ShapeDtypeStruct((N, OUT), jnp.float32)),
)


def _fin_body(acc_ref, cnt_ref, r2_ref, o_ref):
  agg = acc_ref[0, :N, :] + acc_ref[1, :N, :]
  cnt = jnp.sum(cnt_ref[:, :N], axis=0)[:, None]
  o = agg * (1.0 / jnp.maximum(cnt, 1.0)) + r2_ref[...]
  m = jnp.max(o, axis=-1, keepdims=True)
  o_ref[...] = (o - m) - jnp.log(jnp.sum(jnp.exp(o - m), axis=-1,
                                         keepdims=True))


_fin = pl.pallas_call(
    _fin_body,
    out_shape=jax.ShapeDtypeStruct((N, OUT), jnp.float32),
)


def kernel(x, edge_index, W1l, W1r, b1, W2l, W2r, b2):
  src = edge_index[0]
  dst = edge_index[1]
  pad = EPAD - E
  src3 = jnp.concatenate([src, jnp.zeros((pad,), jnp.int32)]).reshape(
      NW, NCH, K)
  src3 = jnp.concatenate([src3, jnp.zeros((NW, 2, K), jnp.int32)], axis=1)
  dst3 = jnp.concatenate([dst, jnp.full((pad,), N, jnp.int32)]).reshape(
      NW, NCH, K)
  dst3 = jnp.concatenate([dst3, jnp.full((NW, 2, K), N, jnp.int32)], axis=1)
  p1, r1 = _mm1(x, W1l, W1r, b1.reshape(1, HID))
  acc1, cnt = _seg_cnt(p1, src3, dst3)
  cnt = cnt.reshape(NC, NPAD)
  p2, r2 = _mid(acc1, cnt, r1, W2l, W2r, b2.reshape(1, OUT))
  acc2 = _seg2(p2, src3, dst3)[0]
  return _fin(acc2, cnt, r2)


# speedup vs baseline: 2.1679x; 2.1679x over previous
import functools

import jax
import jax.numpy as jnp
from jax import lax
from jax.experimental import pallas as pl
from jax.experimental.pallas import tpu as pltpu
from jax.experimental.pallas import tpu_sc as plsc

N = 10000
E = 320000
IN_CH = 128
HID = 128
OUT = 64

NC = 2
NS = 16
NW = NC * NS
K = 128
NCH = -(-E // (NW * K))
EPAD = NW * K * NCH
RPT = 640
NPAD = NS * RPT


def _make_seg(D, with_cnt):
  mesh = plsc.VectorSubcoreMesh(core_axis_name="c", subcore_axis_name="s")
  out_type = [jax.ShapeDtypeStruct((NC, NPAD, D), jnp.float32)]
  scratch = [
      pltpu.VMEM((NCH, K), jnp.int32),
      pltpu.VMEM((NCH, K), jnp.int32),
      pltpu.VMEM((K, D), jnp.float32),
      pltpu.VMEM_SHARED((NPAD, D), jnp.float32),
      pltpu.SemaphoreType.DMA,
  ]
  if with_cnt:
    out_type.append(jax.ShapeDtypeStruct((NC, NS, RPT), jnp.float32))
    scratch += [
        pltpu.VMEM_SHARED((NPAD,), jnp.float32),
        pltpu.VMEM((RPT,), jnp.float32),
        pltpu.VMEM((K,), jnp.float32),
    ]

  def body(p_hbm, src_hbm, dst_hbm, *rest):
    if with_cnt:
      (acc_out, cnt_out, src_v, dst_v, rows_v, acc_sh, sem,
       cnt_sh, zrow_v, ones_v) = rest
    else:
      acc_out, src_v, dst_v, rows_v, acc_sh, sem = rest
    cid = lax.axis_index("c")
    sid = lax.axis_index("s")
    wid = cid * NS + sid
    r0 = sid * RPT

    zv = jnp.zeros((16,), jnp.float32)

    def zb(i, carry):
      for l in range(D // 16):
        rows_v[i, pl.ds(l * 16, 16)] = zv
      return carry
    lax.fori_loop(0, K, zb, 0)
    if with_cnt:
      def cb(i, carry):
        zrow_v[pl.ds(i * 16, 16)] = zv
        return carry
      lax.fori_loop(0, RPT // 16, cb, 0)
      def ob(i, carry):
        ones_v[pl.ds(i * 16, 16)] = zv + 1.0
        return carry
      lax.fori_loop(0, K // 16, ob, 0)
      pltpu.sync_copy(zrow_v, cnt_sh.at[pl.ds(r0, RPT)])

    off = 0
    while off < RPT:
      seg = min(K, RPT - off)
      pltpu.sync_copy(rows_v.at[pl.ds(0, seg)], acc_sh.at[pl.ds(r0 + off, seg)])
      off += seg
    pltpu.sync_copy(src_hbm.at[wid], src_v)
    pltpu.sync_copy(dst_hbm.at[wid], dst_v)
    plsc.subcore_barrier()

    def eb(j, carry):
      pltpu.async_copy(p_hbm.at[src_v.at[j]], rows_v, sem).wait()
      pltpu.sync_copy(rows_v, acc_sh.at[dst_v.at[j]], add=True)
      if with_cnt:
        pltpu.sync_copy(ones_v, cnt_sh.at[dst_v.at[j]], add=True)
      return carry
    lax.fori_loop(0, NCH, eb, 0)
    plsc.subcore_barrier()

    pltpu.sync_copy(acc_sh.at[pl.ds(r0, RPT)], acc_out.at[cid, pl.ds(r0, RPT)])
    if with_cnt:
      pltpu.sync_copy(cnt_sh.at[pl.ds(r0, RPT)], cnt_out.at[cid, sid])

  return pl.kernel(body, out_type=tuple(out_type), mesh=mesh,
                   scratch_types=tuple(scratch),
                   compiler_params=pltpu.CompilerParams(
                       use_tc_tiling_on_sc=False))


_seg_cnt = _make_seg(HID, True)
_seg2 = _make_seg(OUT, False)


def _mm1_body(x_ref, wl_ref, wr_ref, b_ref, p_ref, r_ref):
  x = x_ref[...]
  p_ref[...] = jnp.dot(x, wl_ref[...], preferred_element_type=jnp.float32)
  r_ref[...] = jnp.dot(x, wr_ref[...],
                       preferred_element_type=jnp.float32) + b_ref[...]


_mm1 = pl.pallas_call(
    _mm1_body,
    out_shape=(jax.ShapeDtypeStruct((N, HID), jnp.float32),
               jax.ShapeDtypeStruct((N, HID), jnp.float32)),
)


def _mid_body(acc_ref, cnt_ref, r1_ref, wl_ref, wr_ref, b_ref, p2_ref, r2_ref):
  agg = acc_ref[0, :N, :] + acc_ref[1, :N, :]
  cnt = jnp.sum(cnt_ref[:, :N], axis=0)[:, None]
  rc = 1.0 / jnp.maximum(cnt, 1.0)
  h = jnp.maximum(agg * rc + r1_ref[...], 0.0)
  p2_ref[...] = jnp.dot(h, wl_ref[...], preferred_element_type=jnp.float32)
  r2_ref[...] = jnp.dot(h, wr_ref[...],
                        preferred_element_type=jnp.float32) + b_ref[...]


_mid = pl.pallas_call(
    _mid_body,
    out_shape=(jax.ShapeDtypeStruct((N, OUT), jnp.float32),
               jax.ShapeDtypeStruct((N, OUT), jnp.float32)),
)


def _fin_body(acc_ref, cnt_ref, r2_ref, o_ref):
  agg = acc_ref[0, :N, :] + acc_ref[1, :N, :]
  cnt = jnp.sum(cnt_ref[:, :N], axis=0)[:, None]
  o = agg * (1.0 / jnp.maximum(cnt, 1.0)) + r2_ref[...]
  m = jnp.max(o, axis=-1, keepdims=True)
  o_ref[...] = (o - m) - jnp.log(jnp.sum(jnp.exp(o - m), axis=-1,
                                         keepdims=True))


_fin = pl.pallas_call(
    _fin_body,
    out_shape=jax.ShapeDtypeStruct((N, OUT), jnp.float32),
)


def kernel(x, edge_index, W1l, W1r, b1, W2l, W2r, b2):
  src = edge_index[0]
  dst = edge_index[1]
  pad = EPAD - E
  src3 = jnp.concatenate([src, jnp.zeros((pad,), jnp.int32)]).reshape(
      NW, NCH, K)
  dst3 = jnp.concatenate([dst, jnp.full((pad,), N, jnp.int32)]).reshape(
      NW, NCH, K)
  p1, r1 = _mm1(x, W1l, W1r, b1.reshape(1, HID))
  acc1, cnt = _seg_cnt(p1, src3, dst3)
  cnt = cnt.reshape(NC, NPAD)
  p2, r2 = _mid(acc1, cnt, r1, W2l, W2r, b2.reshape(1, OUT))
  acc2 = _seg2(p2, src3, dst3)[0]
  return _fin(acc2, cnt, r2)
